# baseline (speedup 1.0000x reference)
import functools

import jax
import jax.numpy as jnp
from jax import lax
from jax.experimental import pallas as pl
from jax.experimental.pallas import tpu as pltpu
from jax.experimental.pallas import tpu_sc as plsc

N = 10000
D = 128
E = 320000

NC = 2
NS = 16
NW = NC * NS
CHUNK = 128
N_PAD = 10240
ROWS_PER_TILE = N_PAD // NS
PAD_IDX = N_PAD - 1
CH_PER_TILE = 80
N_CHUNKS = 2560
E_PAD = N_CHUNKS * CHUNK
NBUF = 2
CH0 = 80
CH1 = 80
CH_PHASE = 40
NPHASE0 = CH0 // CH_PHASE
NPHASE1 = CH1 // CH_PHASE

_mesh = plsc.VectorSubcoreMesh(core_axis_name="c", subcore_axis_name="s")


@functools.partial(
    pl.kernel,
    mesh=_mesh,
    out_type=jax.ShapeDtypeStruct((NC, N_PAD), jnp.float32),
    scratch_types=[
        pltpu.VMEM((CH_PER_TILE, 1, CHUNK), jnp.int32),
        pltpu.VMEM((CHUNK,), jnp.float32),
        pltpu.VMEM_SHARED((N_PAD,), jnp.float32),
    ],
)
def _sc_degree(dst_hbm, ones_hbm, zeros_hbm, out_hbm, dst_v, ones_v, acc_sh):
    c = lax.axis_index("c")
    s = lax.axis_index("s")
    pltpu.sync_copy(zeros_hbm, acc_sh.at[pl.ds(s * ROWS_PER_TILE, ROWS_PER_TILE)])
    pltpu.sync_copy(ones_hbm, ones_v)
    base = (c * NS + s) * CH_PER_TILE
    pltpu.sync_copy(dst_hbm.at[pl.ds(base, CH_PER_TILE)], dst_v)
    plsc.subcore_barrier()

    def body(j, carry):
        pltpu.sync_copy(ones_v, acc_sh.at[dst_v.at[j, 0]], add=True)
        return carry

    lax.fori_loop(0, CH_PER_TILE, body, 0)
    plsc.subcore_barrier()
    sl = pl.ds(s * ROWS_PER_TILE, ROWS_PER_TILE)
    pltpu.sync_copy(acc_sh.at[sl], out_hbm.at[c].at[sl])


@functools.partial(
    pl.kernel,
    mesh=_mesh,
    out_type=jax.ShapeDtypeStruct((NC, N_PAD, D), jnp.float32),
    scratch_types=[
        pltpu.VMEM((CH_PHASE, CHUNK), jnp.int32),
        pltpu.VMEM((CH_PHASE, 1, CHUNK), jnp.int32),
        pltpu.VMEM((CHUNK, D), jnp.float32),
        pltpu.VMEM((CHUNK, D), jnp.float32),
        pltpu.VMEM_SHARED((N_PAD, D), jnp.float32),
        pltpu.SemaphoreType.DMA((NBUF,)),
        pltpu.SemaphoreType.DMA((NBUF,)),
    ],
)
def _sc_accum(table_hbm, src_hbm, dst_hbm, out_hbm,
              src_v, dst_v, rb0, rb1, acc_sh, gsem, ssem):
    rows_v = (rb0, rb1)
    c = lax.axis_index("c")
    s = lax.axis_index("s")
    sl = pl.ds(s * ROWS_PER_TILE, ROWS_PER_TILE)

    zv = jnp.zeros((16,), jnp.float32)

    def zrow(i, carry):
        for k in range(D // 16):
            rb0[i, pl.ds(k * 16, 16)] = zv
        return carry

    lax.fori_loop(0, CHUNK, zrow, 0)
    for r in range(ROWS_PER_TILE // CHUNK):
        pltpu.sync_copy(rb0, acc_sh.at[pl.ds(s * ROWS_PER_TILE + r * CHUNK, CHUNK)])

    base = jnp.where(c == 0, s * CH0, NS * CH0 + s * CH1)
    nphase = jnp.where(c == 0, NPHASE0, NPHASE1)
    plsc.subcore_barrier()

    def run_phase(ph, carry):
        pbase = pl.multiple_of(base + ph * CH_PHASE, CH_PHASE)
        pltpu.sync_copy(src_hbm.at[pl.ds(pbase, CH_PHASE)], src_v)
        pltpu.sync_copy(dst_hbm.at[pl.ds(pbase, CH_PHASE)], dst_v)

        for b in range(NBUF):
            pltpu.async_copy(table_hbm.at[src_v.at[b]], rows_v[b], gsem.at[b])

        def outer(j0, carry2):
            for b in range(NBUF):
                j = j0 * NBUF + b
                pltpu.make_async_copy(table_hbm.at[src_v.at[j]], rows_v[b],
                                      gsem.at[b]).wait()
                pltpu.async_copy(rows_v[b], acc_sh.at[dst_v.at[j, 0]],
                                 ssem.at[b], add=True)
                pltpu.make_async_copy(rows_v[b], acc_sh.at[dst_v.at[j, 0]],
                                      ssem.at[b]).wait()

                @pl.when(j + NBUF < CH_PHASE)
                def _():
                    pltpu.async_copy(table_hbm.at[src_v.at[j + NBUF]],
                                     rows_v[b], gsem.at[b])
            return carry2

        lax.fori_loop(0, CH_PHASE // NBUF, outer, 0)
        return carry

    lax.fori_loop(0, nphase, run_phase, 0)
    plsc.subcore_barrier()
    pltpu.sync_copy(acc_sh.at[sl], out_hbm.at[c].at[sl])


def _tc_a0_body(x_ref, w_ref, h_ref):
    h_ref[...] = jnp.dot(x_ref[...], w_ref[...],
                         preferred_element_type=jnp.float32)


def _tc_a1_body(h_ref, degp_ref, hs_ref, dinv_ref):
    deg = degp_ref[0] + degp_ref[1] + 1.0
    dinv = lax.rsqrt(deg)
    hs_ref[...] = h_ref[...] * dinv
    dinv_ref[...] = dinv


def _tc_b_body(acc_ref, h1_ref, dinv_ref, w_ref, b_ref, a_ref, h2_ref, hs2_ref):
    dinv = dinv_ref[...]
    z = dinv * (acc_ref[0] + acc_ref[1]) + (dinv * dinv) * h1_ref[...] + b_ref[...]
    g = jnp.maximum(z, 0.0) + a_ref[0] * jnp.minimum(z, 0.0)
    h2 = jnp.dot(g, w_ref[...], preferred_element_type=jnp.float32)
    h2_ref[...] = h2
    hs2_ref[...] = h2 * dinv


def _tc_c_body(acc_ref, h2_ref, dinv_ref, b_ref, out_ref):
    dinv = dinv_ref[...]
    full = (dinv * (acc_ref[0] + acc_ref[1])
            + (dinv * dinv) * h2_ref[...] + b_ref[...])
    out_ref[...] = full[:N]


_f32 = jnp.float32
_SDS = jax.ShapeDtypeStruct

_tc_a0 = pl.pallas_call(_tc_a0_body, out_shape=_SDS((N_PAD, D), _f32))

_tc_a1 = pl.pallas_call(
    _tc_a1_body,
    out_shape=(_SDS((N_PAD, D), _f32), _SDS((N_PAD, 1), _f32)),
)

_tc_b = pl.pallas_call(
    _tc_b_body,
    out_shape=(_SDS((N_PAD, D), _f32), _SDS((N_PAD, D), _f32)),
)

_tc_c = pl.pallas_call(
    _tc_c_body,
    out_shape=_SDS((N, D), _f32),
)


def kernel(x, edge_index, W1, b1, W2, b2, a):
    ei = edge_index.astype(jnp.int32)
    pad = N + (jnp.arange(E_PAD - E, dtype=jnp.int32) % (N_PAD - N))
    src_p = jnp.concatenate([ei[0], pad]).reshape(N_CHUNKS, CHUNK)
    dst_p = jnp.concatenate([ei[1], pad]).reshape(N_CHUNKS, 1, CHUNK)
    x_pad = jnp.pad(x, ((0, N_PAD - N), (0, 0)))
    ones_c = jnp.ones((CHUNK,), _f32)
    zeros_r1 = jnp.zeros((ROWS_PER_TILE,), _f32)
    b1r = b1.reshape(1, D)
    b2r = b2.reshape(1, D)
    ar = a.reshape(1)

    degp = _sc_degree(dst_p, ones_c, zeros_r1)
    h1 = _tc_a0(x_pad, W1)
    hs1, dinv = _tc_a1(h1, degp.reshape(NC, N_PAD, 1))
    acc1 = _sc_accum(hs1, src_p, dst_p)
    h2, hs2 = _tc_b(acc1, h1, dinv, W2, b1r, ar)
    acc2 = _sc_accum(hs2, src_p, dst_p)
    return _tc_c(acc2, h2, dinv, b2r)


# speedup vs baseline: 1.0984x; 1.0984x over previous
import functools

import jax
import jax.numpy as jnp
from jax import lax
from jax.experimental import pallas as pl
from jax.experimental.pallas import tpu as pltpu
from jax.experimental.pallas import tpu_sc as plsc

N = 10000
D = 128
E = 320000

NC = 2
NS = 16
NW = NC * NS
CHUNK = 128
N_PAD = 10240
ROWS_PER_TILE = N_PAD // NS
PAD_IDX = N_PAD - 1
CH_PER_TILE = 80
N_CHUNKS = 2560
E_PAD = N_CHUNKS * CHUNK
NBUF = 2
CH0 = 80
CH1 = 80
CH_PHASE = 40
NPHASE0 = CH0 // CH_PHASE
NPHASE1 = CH1 // CH_PHASE

_mesh = plsc.VectorSubcoreMesh(core_axis_name="c", subcore_axis_name="s")


@functools.partial(
    pl.kernel,
    mesh=_mesh,
    out_type=jax.ShapeDtypeStruct((NC, N_PAD), jnp.float32),
    scratch_types=[
        pltpu.VMEM((CH_PER_TILE, 1, CHUNK), jnp.int32),
        pltpu.VMEM((CHUNK,), jnp.float32),
        pltpu.VMEM_SHARED((N_PAD,), jnp.float32),
    ],
)
def _sc_degree(dst_hbm, ones_hbm, zeros_hbm, out_hbm, dst_v, ones_v, acc_sh):
    c = lax.axis_index("c")
    s = lax.axis_index("s")
    pltpu.sync_copy(zeros_hbm, acc_sh.at[pl.ds(s * ROWS_PER_TILE, ROWS_PER_TILE)])
    pltpu.sync_copy(ones_hbm, ones_v)
    base = (c * NS + s) * CH_PER_TILE
    pltpu.sync_copy(dst_hbm.at[pl.ds(base, CH_PER_TILE)], dst_v)
    plsc.subcore_barrier()

    def body(j, carry):
        pltpu.sync_copy(ones_v, acc_sh.at[dst_v.at[j, 0]], add=True)
        return carry

    lax.fori_loop(0, CH_PER_TILE, body, 0)
    plsc.subcore_barrier()
    sl = pl.ds(s * ROWS_PER_TILE, ROWS_PER_TILE)
    pltpu.sync_copy(acc_sh.at[sl], out_hbm.at[c].at[sl])


@functools.partial(
    pl.kernel,
    mesh=_mesh,
    out_type=jax.ShapeDtypeStruct((NC, N_PAD, D), jnp.float32),
    scratch_types=[
        pltpu.VMEM((CH_PHASE, CHUNK), jnp.int32),
        pltpu.VMEM((CH_PHASE, 1, CHUNK), jnp.int32),
        pltpu.VMEM((CHUNK, D), jnp.float32),
        pltpu.VMEM((CHUNK, D), jnp.float32),
        pltpu.VMEM_SHARED((N_PAD, D), jnp.float32),
        pltpu.SemaphoreType.DMA((NBUF,)),
        pltpu.SemaphoreType.DMA((NBUF,)),
    ],
)
def _sc_accum(table_hbm, src_hbm, dst_hbm, out_hbm,
              src_v, dst_v, rb0, rb1, acc_sh, gsem, ssem):
    rows_v = (rb0, rb1)
    c = lax.axis_index("c")
    s = lax.axis_index("s")
    sl = pl.ds(s * ROWS_PER_TILE, ROWS_PER_TILE)

    zv = jnp.zeros((16,), jnp.float32)

    def zrow(i, carry):
        for k in range(D // 16):
            rb0[i, pl.ds(k * 16, 16)] = zv
        return carry

    lax.fori_loop(0, CHUNK, zrow, 0)
    for r in range(ROWS_PER_TILE // CHUNK):
        pltpu.sync_copy(rb0, acc_sh.at[pl.ds(s * ROWS_PER_TILE + r * CHUNK, CHUNK)])

    base = jnp.where(c == 0, s * CH0, NS * CH0 + s * CH1)
    nphase = jnp.where(c == 0, NPHASE0, NPHASE1)
    plsc.subcore_barrier()

    def run_phase(ph, carry):
        pbase = pl.multiple_of(base + ph * CH_PHASE, CH_PHASE)
        pltpu.sync_copy(src_hbm.at[pl.ds(pbase, CH_PHASE)], src_v)
        pltpu.sync_copy(dst_hbm.at[pl.ds(pbase, CH_PHASE)], dst_v)

        for b in range(NBUF):
            pltpu.async_copy(table_hbm.at[src_v.at[b]], rows_v[b], gsem.at[b])

        def outer(j0, carry2):
            for b in range(NBUF):
                j = j0 * NBUF + b
                pltpu.make_async_copy(table_hbm.at[src_v.at[j]], rows_v[b],
                                      gsem.at[b]).wait()
                pltpu.async_copy(rows_v[b], acc_sh.at[dst_v.at[j, 0]],
                                 ssem.at[b], add=True)
                pltpu.make_async_copy(rows_v[b], acc_sh.at[dst_v.at[j, 0]],
                                      ssem.at[b]).wait()

                @pl.when(j + NBUF < CH_PHASE)
                def _():
                    pltpu.async_copy(table_hbm.at[src_v.at[j + NBUF]],
                                     rows_v[b], gsem.at[b])
            return carry2

        lax.fori_loop(0, CH_PHASE // NBUF, outer, 0)
        return carry

    lax.fori_loop(0, nphase, run_phase, 0)
    plsc.subcore_barrier()
    pltpu.sync_copy(acc_sh.at[sl], out_hbm.at[c].at[sl])


def _tc_a0_body(x_ref, w_ref, h_ref):
    h_ref[...] = jnp.dot(x_ref[...], w_ref[...],
                         preferred_element_type=jnp.float32)


def _tc_a1_body(h_ref, degp_ref, hs_ref, dinv_ref):
    deg = degp_ref[0] + degp_ref[1] + 1.0
    dinv = lax.rsqrt(deg)
    hs_ref[...] = h_ref[...] * dinv
    dinv_ref[...] = dinv


def _tc_b_body(acc_ref, h1_ref, dinv_ref, w_ref, b_ref, a_ref, h2_ref, hs2_ref):
    dinv = dinv_ref[...]
    z = dinv * (acc_ref[0] + acc_ref[1]) + (dinv * dinv) * h1_ref[...] + b_ref[...]
    g = jnp.maximum(z, 0.0) + a_ref[0] * jnp.minimum(z, 0.0)
    h2 = jnp.dot(g, w_ref[...], preferred_element_type=jnp.float32)
    h2_ref[...] = h2
    hs2_ref[...] = h2 * dinv


def _tc_c_body(acc_ref, h2_ref, dinv_ref, b_ref, out_ref):
    dinv = dinv_ref[...]
    out_ref[...] = (dinv * (acc_ref[0] + acc_ref[1])
                    + (dinv * dinv) * h2_ref[...] + b_ref[...])


_f32 = jnp.float32
_SDS = jax.ShapeDtypeStruct

_tc_a0 = pl.pallas_call(_tc_a0_body, out_shape=_SDS((N_PAD, D), _f32))

_tc_a1 = pl.pallas_call(
    _tc_a1_body,
    out_shape=(_SDS((N_PAD, D), _f32), _SDS((N_PAD, 1), _f32)),
)

_tc_b = pl.pallas_call(
    _tc_b_body,
    out_shape=(_SDS((N_PAD, D), _f32), _SDS((N_PAD, D), _f32)),
)

_tc_c = pl.pallas_call(
    _tc_c_body,
    out_shape=_SDS((N_PAD, D), _f32),
)


def kernel(x, edge_index, W1, b1, W2, b2, a):
    ei = edge_index.astype(jnp.int32)
    pad = N + (jnp.arange(E_PAD - E, dtype=jnp.int32) % (N_PAD - N))
    src_p = jnp.concatenate([ei[0], pad]).reshape(N_CHUNKS, CHUNK)
    dst_p = jnp.concatenate([ei[1], pad]).reshape(N_CHUNKS, 1, CHUNK)
    x_pad = jnp.pad(x, ((0, N_PAD - N), (0, 0)))
    ones_c = jnp.ones((CHUNK,), _f32)
    zeros_r1 = jnp.zeros((ROWS_PER_TILE,), _f32)
    b1r = b1.reshape(1, D)
    b2r = b2.reshape(1, D)
    ar = a.reshape(1)

    degp = _sc_degree(dst_p, ones_c, zeros_r1)
    h1 = _tc_a0(x_pad, W1)
    hs1, dinv = _tc_a1(h1, degp.reshape(NC, N_PAD, 1))
    acc1 = _sc_accum(hs1, src_p, dst_p)
    h2, hs2 = _tc_b(acc1, h1, dinv, W2, b1r, ar)
    acc2 = _sc_accum(hs2, src_p, dst_p)
    out = _tc_c(acc2, h2, dinv, b2r)
    return out[:N]
